# baseline (speedup 1.0000x reference)
import jax
import jax.numpy as jnp
from jax.experimental import pallas as pl

K = 1024
D = 64
N = 32768
BLK = 1024
NB = N // BLK


def _kmeans_step_body(xa_ref, ct_ref, out_ref):
    i = pl.program_id(0)
    xa = xa_ref[...]
    x = xa[:, :D]
    ct = ct_ref[...]
    cnorm = jnp.sum(ct * ct, axis=0)
    scores = jax.lax.dot_general(
        x, ct, (((1,), (0,)), ((), ())),
        preferred_element_type=jnp.float32,
        precision=jax.lax.Precision.HIGHEST,
    )
    dist = cnorm[None, :] - 2.0 * scores
    idx = jnp.argmin(dist, axis=1).astype(jnp.int32)
    onehot_t = (idx[None, :] ==
                jax.lax.broadcasted_iota(jnp.int32, (K, BLK), 0)
                ).astype(jnp.float32)
    part = jax.lax.dot_general(
        onehot_t, xa, (((1,), (0,)), ((), ())),
        preferred_element_type=jnp.float32,
        precision=jax.lax.Precision.HIGHEST,
    )

    @pl.when(i == 0)
    def _init():
        out_ref[...] = part

    @pl.when(i > 0)
    def _acc():
        out_ref[...] += part


@jax.jit
def _kmeans_step(xa, ct):
    out = pl.pallas_call(
        _kmeans_step_body,
        grid=(NB,),
        in_specs=[
            pl.BlockSpec((BLK, D + 1), lambda i: (i, 0)),
            pl.BlockSpec((D, K), lambda i: (0, 0)),
        ],
        out_specs=pl.BlockSpec((K, D + 1), lambda i: (0, 0)),
        out_shape=jax.ShapeDtypeStruct((K, D + 1), jnp.float32),
    )(xa, ct)
    return out


def kernel(locF, Ck):
    x = locF.reshape(N, D)
    xa = jnp.concatenate([x, jnp.ones((N, 1), jnp.float32)], axis=1)
    ct = Ck.reshape(K, D).T
    out = _kmeans_step(xa, ct)
    Ck1 = out[None, :, :D]
    nItems = out[:, D][None, :].astype(jnp.int64)
    return (Ck1, nItems)


# speedup vs baseline: 15.8045x; 15.8045x over previous
import functools

import jax
import jax.numpy as jnp
from jax import lax
from jax.experimental import pallas as pl
from jax.experimental.pallas import tpu as pltpu
from jax.experimental.pallas import tpu_sc as plsc

K = 1024
D = 64
N = 32768
BLK = 1024
NB = N // BLK

_DN = (((1,), (0,)), ((), ()))
_f32 = jnp.float32
_i32 = jnp.int32


def _mm(a, b):
    return jax.lax.dot_general(a, b, _DN, preferred_element_type=_f32)


def _assign_body(x0_ref, x1_ref, x2_ref, c0_ref, c1_ref, c2_ref, idx_ref):
    x0, x1, x2 = x0_ref[...], x1_ref[...], x2_ref[...]
    c0, c1, c2 = c0_ref[...], c1_ref[...], c2_ref[...]
    ctf = (c0.astype(_f32) + c1.astype(_f32) + c2.astype(_f32))
    cnorm = jnp.sum(ctf * ctf, axis=0)[None, :]
    scores = (_mm(x0, c0)
              + (_mm(x0, c1) + _mm(x1, c0))
              + (_mm(x0, c2) + _mm(x1, c1) + _mm(x2, c0)))
    dist = cnorm - 2.0 * scores
    idx = jnp.argmin(dist, axis=1).astype(_i32)
    idx_ref[...] = idx[None, None, :]


@jax.jit
def _assign(x0, x1, x2, c0, c1, c2):
    xspec = pl.BlockSpec((BLK, D), lambda i: (i, 0))
    cspec = pl.BlockSpec((D, K), lambda i: (0, 0))
    idx = pl.pallas_call(
        _assign_body,
        grid=(NB,),
        in_specs=[xspec, xspec, xspec, cspec, cspec, cspec],
        out_specs=pl.BlockSpec((1, 1, BLK), lambda i: (i, 0, 0)),
        out_shape=jax.ShapeDtypeStruct((NB, 1, BLK), _i32),
    )(x0, x1, x2, c0, c1, c2)
    return idx


def _split3(m):
    h = m.astype(jnp.bfloat16)
    r = m - h.astype(_f32)
    mid = r.astype(jnp.bfloat16)
    lo = (r - mid.astype(_f32)).astype(jnp.bfloat16)
    return h, mid, lo


NC = 2
NS = 16
KH = K // NC
ACC_ROWS = KH + 16
ZR = ACC_ROWS // NS
RPT = N // NS
CH = 128
NCH = RPT // CH
OR_ = KH // NS


def _sc_scatter_body(x_hbm, idx_hbm, zall_hbm, ones_hbm,
                     sums_hbm, counts_hbm,
                     idx_raw, idx_adj, idx_adjc, x_buf, ones_buf, acc):
    c = lax.axis_index("c")
    s = lax.axis_index("s")
    base_k = c * KH

    @pl.when(s == 0)
    def _zero():
        pltpu.sync_copy(zall_hbm, acc)

    pltpu.sync_copy(ones_hbm, ones_buf)

    plsc.subcore_barrier()

    for j in range(NCH):
        row0 = s * RPT + j * CH
        pltpu.sync_copy(idx_hbm.at[pl.ds(row0, CH)], idx_raw)
        for g in range(CH // 16):
            v = idx_raw[pl.ds(g * 16, 16)]
            vl = v - base_k
            in_half = (vl >= 0) & (vl < KH)
            adj = jnp.where(in_half, vl, KH + (v & 15))
            idx_adj[pl.ds(g * 16, 16)] = adj
            idx_adjc[pl.ds(g * 16, 16)] = adj + ACC_ROWS
        pltpu.sync_copy(x_hbm.at[pl.ds(row0, CH)], x_buf)
        pltpu.sync_copy(x_buf, acc.at[idx_adj], add=True)
        pltpu.sync_copy(ones_buf, acc.at[idx_adjc], add=True)

    plsc.subcore_barrier()

    @pl.when((s == 0) & (c == 0))
    def _out0():
        pltpu.sync_copy(acc.at[pl.ds(0, KH)], sums_hbm.at[pl.ds(0, KH)])
        pltpu.sync_copy(acc.at[pl.ds(ACC_ROWS, KH)], counts_hbm.at[pl.ds(0, KH)])

    @pl.when((s == 0) & (c == 1))
    def _out1():
        pltpu.sync_copy(acc.at[pl.ds(0, KH)], sums_hbm.at[pl.ds(KH, KH)])
        pltpu.sync_copy(acc.at[pl.ds(ACC_ROWS, KH)], counts_hbm.at[pl.ds(KH, KH)])


@jax.jit
def _sc_scatter(x, idx):
    mesh = plsc.VectorSubcoreMesh(core_axis_name="c", subcore_axis_name="s")
    kfn = functools.partial(
        pl.kernel,
        mesh=mesh,
        out_type=[
            jax.ShapeDtypeStruct((K, 128), _f32),
            jax.ShapeDtypeStruct((K, 128), _f32),
        ],
        scratch_types=[
            pltpu.VMEM((CH,), _i32),
            pltpu.VMEM((CH,), _i32),
            pltpu.VMEM((CH,), _i32),
            pltpu.VMEM((CH, 128), _f32),
            pltpu.VMEM((CH, 128), _f32),
            pltpu.VMEM_SHARED((2 * ACC_ROWS, 128), _f32),
        ],
    )(_sc_scatter_body)
    x_pad = jnp.pad(x, ((0, 0), (0, 128 - D)))
    zall = jnp.zeros((2 * ACC_ROWS, 128), _f32)
    ones = jnp.ones((CH, 128), _f32)
    return kfn(x_pad, idx, zall, ones)


def kernel(locF, Ck):
    x = locF.reshape(N, D)
    ct = Ck.reshape(K, D).T
    x0, x1, x2 = _split3(x)
    c0, c1, c2 = _split3(ct)
    idx = _assign(x0, x1, x2, c0, c1, c2).reshape(N)
    sums128, counts128 = _sc_scatter(x, idx)
    Ck1 = sums128[:, :D][None, :, :]
    nItems = counts128[:, 0][None, :].astype(jnp.int64)
    return (Ck1, nItems)


# speedup vs baseline: 16.4878x; 1.0432x over previous
import functools

import jax
import jax.numpy as jnp
from jax import lax
from jax.experimental import pallas as pl
from jax.experimental.pallas import tpu as pltpu
from jax.experimental.pallas import tpu_sc as plsc

K = 1024
D = 64
N = 32768
BLK = 1024
NB = N // BLK

_DN = (((1,), (0,)), ((), ()))
_f32 = jnp.float32
_i32 = jnp.int32


def _mm(a, b):
    return jax.lax.dot_general(a, b, _DN, preferred_element_type=_f32)


def _split3(m):
    h = m.astype(jnp.bfloat16)
    r = m - h.astype(_f32)
    mid = r.astype(jnp.bfloat16)
    lo = (r - mid.astype(_f32)).astype(jnp.bfloat16)
    return h, mid, lo


def _assign_body(x_ref, ct_ref, idx_ref):
    x = x_ref[...]
    ct = ct_ref[...]
    cnorm = jnp.sum(ct * ct, axis=0)[None, :]
    x0, x1, x2 = _split3(x)
    c0, c1, c2 = _split3(ct)
    scores = (_mm(x0, c0)
              + (_mm(x0, c1) + _mm(x1, c0))
              + (_mm(x0, c2) + _mm(x1, c1) + _mm(x2, c0)))
    dist = cnorm - 2.0 * scores
    idx = jnp.argmin(dist, axis=1).astype(_i32)
    idx_ref[...] = idx[None, None, :]


@jax.jit
def _assign(x, ct):
    idx = pl.pallas_call(
        _assign_body,
        grid=(NB,),
        in_specs=[
            pl.BlockSpec((BLK, D), lambda i: (i, 0)),
            pl.BlockSpec((D, K), lambda i: (0, 0)),
        ],
        out_specs=pl.BlockSpec((1, 1, BLK), lambda i: (i, 0, 0)),
        out_shape=jax.ShapeDtypeStruct((NB, 1, BLK), _i32),
    )(x, ct)
    return idx


NC = 2
NS = 16
KH = K // NC
ACC_ROWS = KH + 16
ZR = ACC_ROWS // NS
RPT = N // NS
CH = 128
NCH = RPT // CH
OR_ = KH // NS


def _sc_scatter_body(x_hbm, idx_hbm, zall_hbm, ones_hbm,
                     sums_hbm, counts_hbm,
                     idx_raw, idx_adj, idx_adjc, x_buf, ones_buf, acc):
    c = lax.axis_index("c")
    s = lax.axis_index("s")
    base_k = c * KH

    @pl.when(s == 0)
    def _zero():
        pltpu.sync_copy(zall_hbm, acc)

    pltpu.sync_copy(ones_hbm, ones_buf)

    plsc.subcore_barrier()

    for j in range(NCH):
        row0 = s * RPT + j * CH
        pltpu.sync_copy(idx_hbm.at[pl.ds(row0, CH)], idx_raw)
        for g in range(CH // 16):
            v = idx_raw[pl.ds(g * 16, 16)]
            vl = v - base_k
            in_half = (vl >= 0) & (vl < KH)
            adj = jnp.where(in_half, vl, KH + (v & 15))
            idx_adj[pl.ds(g * 16, 16)] = adj
            idx_adjc[pl.ds(g * 16, 16)] = adj + ACC_ROWS
        pltpu.sync_copy(x_hbm.at[pl.ds(row0, CH)], x_buf)
        pltpu.sync_copy(x_buf, acc.at[idx_adj], add=True)
        pltpu.sync_copy(ones_buf, acc.at[idx_adjc], add=True)

    plsc.subcore_barrier()

    @pl.when((s == 0) & (c == 0))
    def _out0():
        pltpu.sync_copy(acc.at[pl.ds(0, KH)], sums_hbm.at[pl.ds(0, KH)])
        pltpu.sync_copy(acc.at[pl.ds(ACC_ROWS, KH)], counts_hbm.at[pl.ds(0, KH)])

    @pl.when((s == 0) & (c == 1))
    def _out1():
        pltpu.sync_copy(acc.at[pl.ds(0, KH)], sums_hbm.at[pl.ds(KH, KH)])
        pltpu.sync_copy(acc.at[pl.ds(ACC_ROWS, KH)], counts_hbm.at[pl.ds(KH, KH)])


@jax.jit
def _sc_scatter(x, idx):
    mesh = plsc.VectorSubcoreMesh(core_axis_name="c", subcore_axis_name="s")
    kfn = functools.partial(
        pl.kernel,
        mesh=mesh,
        out_type=[
            jax.ShapeDtypeStruct((K, 128), _f32),
            jax.ShapeDtypeStruct((K, 128), _f32),
        ],
        scratch_types=[
            pltpu.VMEM((CH,), _i32),
            pltpu.VMEM((CH,), _i32),
            pltpu.VMEM((CH,), _i32),
            pltpu.VMEM((CH, 128), _f32),
            pltpu.VMEM((CH, 128), _f32),
            pltpu.VMEM_SHARED((2 * ACC_ROWS, 128), _f32),
        ],
    )(_sc_scatter_body)
    x_pad = jnp.pad(x, ((0, 0), (0, 128 - D)))
    zall = jnp.zeros((2 * ACC_ROWS, 128), _f32)
    ones = jnp.ones((CH, 128), _f32)
    x_pad, idx, zall, ones = jax.lax.optimization_barrier(
        (x_pad, idx, zall, ones))
    return kfn(x_pad, idx, zall, ones)


def kernel(locF, Ck):
    x = locF.reshape(N, D)
    ct = Ck.reshape(K, D).T
    idx = _assign(x, ct).reshape(N)
    sums128, counts128 = _sc_scatter(x, idx)
    Ck1 = sums128[:, :D][None, :, :]
    nItems = counts128[:, 0][None, :].astype(jnp.int64)
    return (Ck1, nItems)
